# baseline (speedup 1.0000x reference)
import jax
import jax.numpy as jnp
from jax import lax
from jax.experimental import pallas as pl
from jax.experimental.pallas import tpu as pltpu

N0 = 128
F1 = 8
F2 = 8
D = 6424
H = 256
B = 8
NSTEP = N0 // B
E = B * F1
R = E * F2
P = 4
RP = R // P


def _fused_body(x0_ref, x1_ref, x2_hbm, wbig_ref, wa0_ref,
                ws1_ref, wa1_ref, w1top_ref, b1_ref, lng_ref, lnb_ref,
                w2_ref, b2_ref, out_ref, xbuf, sems):
    f32 = jnp.float32
    bf16 = jnp.bfloat16
    i = pl.program_id(0)

    def slab_copy(step, slot, p):
        return pltpu.make_async_copy(
            x2_hbm.at[pl.ds(step * R + p * RP, RP), :],
            xbuf.at[slot, pl.ds(p * RP, RP), :],
            sems.at[slot, p])

    @pl.when(i == 0)
    def _():
        for p in range(P):
            slab_copy(0, 0, p).start()

    @pl.when(i + 1 < NSTEP)
    def _():
        nxt = lax.rem(i + 1, 2)
        for p in range(P):
            slab_copy(i + 1, nxt, p).start()

    slot = lax.rem(i, 2)
    for p in range(P):
        slab_copy(i, slot, p).wait()

    x1b = x1_ref[...]
    xs1 = x1b.reshape(E, D)
    m1 = x1b.sum(axis=1) * (1.0 / F1)

    x2b = xbuf[slot].reshape(E, F2, D)
    m2 = x2b[:, 0, :]
    for j in range(1, F2):
        m2 = m2 + x2b[:, j, :]
    m2 = m2 * (1.0 / F2)

    wbig = wbig_ref[...]
    ws0 = wbig[:, :H]
    wa0 = wa0_ref[...]

    h0 = jnp.maximum(
        jnp.dot(x0_ref[...].astype(bf16), ws0, preferred_element_type=f32)
        + jnp.dot(m1.astype(bf16), wa0, preferred_element_type=f32), 0.0)
    big = jnp.dot(xs1.astype(bf16), wbig, preferred_element_type=f32)
    h1 = jnp.maximum(
        big[:, :H] + jnp.dot(m2.astype(bf16), wa0,
                             preferred_element_type=f32), 0.0)

    mh1 = h1.reshape(B, F1, H).sum(axis=1) * (1.0 / F1)
    g0 = (jnp.dot(h0, ws1_ref[...], preferred_element_type=f32)
          + jnp.dot(mh1, wa1_ref[...], preferred_element_type=f32))
    t = jnp.dot(g0, w1top_ref[...], preferred_element_type=f32)
    trep = jnp.broadcast_to(t[:, None, :], (B, F1, H)).reshape(E, H)

    e = big[:, H:] + trep + b1_ref[...]
    mu = e.mean(axis=-1, keepdims=True)
    var = ((e - mu) ** 2).mean(axis=-1, keepdims=True)
    hn = (e - mu) * jax.lax.rsqrt(var + 1e-5) * lng_ref[...] + lnb_ref[...]
    hn = jnp.maximum(hn, 0.0)
    out_ref[...] = (jnp.dot(hn, w2_ref[...], preferred_element_type=f32)
                    + b2_ref[...])


def kernel(x0, x1, x2, W_self0, W_agg0, W_self1, W_agg1,
           mlp_w1, mlp_b1, ln_g, ln_b, mlp_w2, mlp_b2):
    x1v = x1.reshape(N0, F1, D)
    bf16 = jnp.bfloat16
    wbig = jnp.concatenate([W_self0, mlp_w1[H:]], axis=1).astype(bf16)
    wa0 = W_agg0.astype(bf16)
    w1top = mlp_w1[:H]
    b1 = mlp_b1.reshape(1, H)
    lng = ln_g.reshape(1, H)
    lnb = ln_b.reshape(1, H)
    b2 = mlp_b2.reshape(1, 1)

    full = lambda shape: pl.BlockSpec(shape, lambda i: (0,) * len(shape))
    out = pl.pallas_call(
        _fused_body,
        grid=(NSTEP,),
        in_specs=[
            pl.BlockSpec((B, D), lambda i: (i, 0)),
            pl.BlockSpec((B, F1, D), lambda i: (i, 0, 0)),
            pl.BlockSpec(memory_space=pl.ANY),
            full((D, 2 * H)),
            full((D, H)),
            full((H, H)),
            full((H, H)),
            full((H, H)),
            full((1, H)),
            full((1, H)),
            full((1, H)),
            full((H, 1)),
            full((1, 1)),
        ],
        out_specs=pl.BlockSpec((E, 1), lambda i: (i, 0)),
        out_shape=jax.ShapeDtypeStruct((N0 * F1, 1), jnp.float32),
        scratch_shapes=[
            pltpu.VMEM((2, R, D), jnp.float32),
            pltpu.SemaphoreType.DMA((2, P)),
        ],
        compiler_params=pltpu.CompilerParams(
            dimension_semantics=("arbitrary",),
        ),
    )(x0, x1v, x2, wbig, wa0, W_self1, W_agg1, w1top, b1, lng, lnb,
      mlp_w2, b2)
    return out
